# baseline (speedup 1.0000x reference)
import functools

import jax
import jax.numpy as jnp
from jax import lax
from jax.experimental import pallas as pl
from jax.experimental.pallas import tpu as pltpu
from jax.experimental.pallas import tpu_sc as plsc

B = 128
P = 16
HP = 14
SEQ = HP * HP
D = 768
FAN = 768
KEEP = 49
BM = 4
ROWS = BM * (KEEP + 1)
NW = 32
SPW = B // NW
MAPW = HP * P
NB = 3


def _rank_kernel(noise_ref, mask_ref, ids_ref, map_ref):
    n = noise_ref[0]
    jj2 = jax.lax.broadcasted_iota(jnp.int32, (SEQ, SEQ), 0)
    ii2 = jax.lax.broadcasted_iota(jnp.int32, (SEQ, SEQ), 1)
    pr = jax.lax.broadcasted_iota(jnp.int32, (SEQ, MAPW), 0)
    pc = jax.lax.broadcasted_iota(jnp.int32, (SEQ, MAPW), 1)
    perm = jnp.where(pr == (pc // P) * HP + pc % P, 1.0, 0.0)
    jl = jax.lax.broadcasted_iota(jnp.int32, (1, MAPW), 1) % P
    rank_rows = []
    map_rows = []
    for bi in range(BM):
        nb = n[bi:bi + 1, :]
        nbt = jnp.transpose(nb)
        cmp2 = (nbt < nb) | ((nbt == nb) & (jj2 < ii2))
        rank_b = jnp.sum(cmp2.astype(jnp.float32), axis=0, keepdims=True)
        rank_rows.append(rank_b)
        m = jnp.dot(rank_b, perm, preferred_element_type=jnp.float32)
        keep = (jl < HP) & (m < float(KEEP))
        map_rows.append(jnp.where(keep, m, -1.0))
    rank = jnp.concatenate(rank_rows, axis=0)
    ids_ref[0] = rank.astype(jnp.int32)
    mask_ref[0] = jnp.where(rank >= float(KEEP), 1.0, 0.0)
    map_ref[0] = jnp.concatenate(map_rows, axis=0).astype(jnp.int32)


def _sc_gather(map_hbm, pix_hbm, x_hbm, map_v, band_v, sp_rows, sem):
    wid = lax.axis_index("s") * 2 + lax.axis_index("c")
    sid = lax.axis_index("s")

    def sample_body(t, carry):
        b = wid * SPW + t
        pltpu.sync_copy(map_hbm.at[b], map_v)

        def fire(pi):
            return pltpu.async_copy(
                pix_hbm.at[b, :, pl.ds(pi * P, P), :],
                band_v.at[pi % NB], sem)

        cps = [fire(pi) for pi in range(NB)]
        for pi in range(HP):
            cps[pi % NB].wait()
            chunk = map_v[pl.ds(pi * P, P)]
            for pj in range(HP):
                slot = chunk[pj]

                @pl.when(slot >= 0)
                def _():
                    pltpu.sync_copy(
                        band_v.at[pi % NB, :, :, pl.ds(pj * P, P)],
                        sp_rows.at[sid, slot])
            if pi + NB < HP:
                cps[(pi + NB) % NB] = fire(pi + NB)
        pltpu.sync_copy(sp_rows.at[sid],
                        x_hbm.at[pl.ds(b * (KEEP + 1) + 1, KEEP)])
        return carry

    lax.fori_loop(0, SPW, sample_body, 0)


def _emb_kernel(ids_ref, x_ref, w_ref, pos1_ref, b_ref,
                cls_ref, pos0_ref, emb_ref):
    rank = ids_ref[0].astype(jnp.float32)
    r_i = jax.lax.broadcasted_iota(jnp.int32, (ROWS, SEQ), 0)
    rb = r_i // (KEEP + 1)
    rk = r_i % (KEEP + 1)
    target = (rk - 1).astype(jnp.float32)
    posg = jnp.zeros((ROWS, D), jnp.float32)
    for cb in range(BM):
        rank_b = rank[cb:cb + 1, :]
        cond = (rb == cb) & (rk >= 1) & (rank_b == target)
        strip = jnp.where(cond, 1.0, 0.0)
        posg = posg + jnp.dot(strip, pos1_ref[...],
                              preferred_element_type=jnp.float32)
    y = jax.lax.dot_general(
        x_ref[...], w_ref[...], (((1,), (1,)), ((), ())),
        preferred_element_type=jnp.float32)
    out = y + posg + b_ref[...]
    cls_row = cls_ref[...] + pos0_ref[...]
    rr = jax.lax.broadcasted_iota(jnp.int32, (ROWS, D), 0)
    emb_ref[...] = jnp.where(rr % (KEEP + 1) == 0, cls_row, out)


def kernel(pixel_values, noise, W, b, cls_token, pos_embed):
    noise3 = noise.reshape(B // BM, BM, SEQ)
    pos1 = pos_embed[0, 1:, :]
    pos0 = pos_embed[0, :1, :]
    cls2 = cls_token[0]
    b2 = b[None, :]

    grid = (B // BM,)
    mask3, ids3, map3 = pl.pallas_call(
        _rank_kernel,
        grid=grid,
        in_specs=[pl.BlockSpec((1, BM, SEQ), lambda i: (i, 0, 0))],
        out_specs=[
            pl.BlockSpec((1, BM, SEQ), lambda i: (i, 0, 0)),
            pl.BlockSpec((1, BM, SEQ), lambda i: (i, 0, 0)),
            pl.BlockSpec((1, BM, MAPW), lambda i: (i, 0, 0)),
        ],
        out_shape=[
            jax.ShapeDtypeStruct((B // BM, BM, SEQ), jnp.float32),
            jax.ShapeDtypeStruct((B // BM, BM, SEQ), jnp.int32),
            jax.ShapeDtypeStruct((B // BM, BM, MAPW), jnp.int32),
        ],
    )(noise3)

    smap = map3.reshape(B, MAPW)

    sc = functools.partial(
        pl.kernel,
        mesh=plsc.VectorSubcoreMesh(core_axis_name="c", subcore_axis_name="s"),
        compiler_params=pltpu.CompilerParams(use_tc_tiling_on_sc=False),
        out_type=jax.ShapeDtypeStruct((B * (KEEP + 1), 3, P, P), jnp.float32),
        scratch_types=[
            pltpu.VMEM((MAPW,), jnp.int32),
            pltpu.VMEM((NB, 3, P, 224), jnp.float32),
            pltpu.VMEM_SHARED((16, KEEP, 3, P, P), jnp.float32),
            pltpu.SemaphoreType.DMA,
        ],
    )(_sc_gather)
    x4 = sc(smap, pixel_values)
    x_slot = x4.reshape(B * (KEEP + 1), D)

    emb_flat = pl.pallas_call(
        _emb_kernel,
        grid=grid,
        in_specs=[
            pl.BlockSpec((1, BM, SEQ), lambda i: (i, 0, 0)),
            pl.BlockSpec((ROWS, D), lambda i: (i, 0)),
            pl.BlockSpec((D, FAN), lambda i: (0, 0)),
            pl.BlockSpec((SEQ, D), lambda i: (0, 0)),
            pl.BlockSpec((1, D), lambda i: (0, 0)),
            pl.BlockSpec((1, D), lambda i: (0, 0)),
            pl.BlockSpec((1, D), lambda i: (0, 0)),
        ],
        out_specs=pl.BlockSpec((ROWS, D), lambda i: (i, 0)),
        out_shape=jax.ShapeDtypeStruct((B * (KEEP + 1), D), jnp.float32),
    )(ids3, x_slot, W, pos1, b2, cls2, pos0)

    embeddings = emb_flat.reshape(B, KEEP + 1, D)
    mask = mask3.reshape(B, SEQ)
    ids_restore = ids3.reshape(B, SEQ)
    return (embeddings, mask, ids_restore)
